# baseline (speedup 1.0000x reference)
import functools

import jax
import jax.numpy as jnp
from jax.experimental import pallas as pl
from jax.experimental.pallas import tpu as pltpu

_NBUF = 4
_LOOKAHEAD = 3
_CHUNK = 16384


def _make_body(chunk, last, nsteps, valid_last):
    def body(q_ref, kt_hbm, vt_hbm, h_ref, o_ref,
             k_buf, v_buf, qn_ref, m_ref, d_ref, acc_ref, k_sem, v_sem):
        i = pl.program_id(0)

        def start(step):
            step = jnp.asarray(step, jnp.int32)
            b = jax.lax.rem(step, _NBUF)

            @pl.when(step < nsteps - 1)
            def _full():
                cols = pl.ds(step * chunk, chunk)
                pltpu.make_async_copy(
                    kt_hbm.at[:, cols], k_buf.at[b], k_sem.at[b]).start()
                pltpu.make_async_copy(
                    vt_hbm.at[:, cols], v_buf.at[b], v_sem.at[b]).start()

            @pl.when(step == nsteps - 1)
            def _tail():
                cols = pl.ds(step * chunk, last)
                dst = pl.ds(0, last)
                pltpu.make_async_copy(
                    kt_hbm.at[:, cols], k_buf.at[b, :, dst], k_sem.at[b]).start()
                pltpu.make_async_copy(
                    vt_hbm.at[:, cols], v_buf.at[b, :, dst], v_sem.at[b]).start()

        @pl.when(i == 0)
        def _init():
            for sstep in range(min(_LOOKAHEAD, nsteps)):
                start(sstep)
            q = q_ref[...]
            qn = q / jnp.maximum(
                jnp.sqrt(jnp.sum(q * q, axis=1, keepdims=True)), 1e-12)
            qn_ref[...] = qn
            m_ref[...] = jnp.full_like(m_ref, -jnp.inf)
            d_ref[...] = jnp.zeros_like(d_ref)
            acc_ref[...] = jnp.zeros_like(acc_ref)

        @pl.when(jnp.logical_and(i > 0, i + _LOOKAHEAD - 1 < nsteps))
        def _prefetch():
            start(i + _LOOKAHEAD - 1)

        b = jax.lax.rem(i, _NBUF)

        @pl.when(i < nsteps - 1)
        def _wait_full():
            cols = pl.ds(i * chunk, chunk)
            pltpu.make_async_copy(
                kt_hbm.at[:, cols], k_buf.at[b], k_sem.at[b]).wait()
            pltpu.make_async_copy(
                vt_hbm.at[:, cols], v_buf.at[b], v_sem.at[b]).wait()

        @pl.when(i == nsteps - 1)
        def _wait_tail():
            cols = pl.ds(i * chunk, last)
            dst = pl.ds(0, last)
            pltpu.make_async_copy(
                kt_hbm.at[:, cols], k_buf.at[b, :, dst], k_sem.at[b]).wait()
            pltpu.make_async_copy(
                vt_hbm.at[:, cols], v_buf.at[b, :, dst], v_sem.at[b]).wait()

        kt = k_buf[b]
        qn = qn_ref[...]
        d = qn.shape[1]
        raw = jax.lax.dot_general(
            qn, kt, (((1,), (0,)), ((), ())),
            preferred_element_type=jnp.float32)
        ones = jnp.ones((1, d), jnp.float32)
        sumsq = jax.lax.dot_general(
            ones, kt * kt, (((1,), (0,)), ((), ())),
            preferred_element_type=jnp.float32)
        inv_norm = 1.0 / jnp.maximum(jnp.sqrt(sumsq), 1e-12)
        scores = raw * (h_ref[0] * inv_norm * 10.0)
        v = v_buf[b]
        col = jax.lax.broadcasted_iota(jnp.int32, (1, chunk), 1)
        bad = jnp.logical_and(i == nsteps - 1, col >= valid_last)
        scores = jnp.where(bad, -1e30, scores)
        v = jnp.where(bad, 0.0, v)

        m_prev = m_ref[...]
        m_new = jnp.maximum(m_prev, jnp.max(scores, axis=1, keepdims=True))
        alpha = jnp.exp(m_prev - m_new)
        p = jnp.exp(scores - m_new)
        m_ref[...] = m_new
        d_ref[...] = d_ref[...] * alpha + jnp.sum(p, axis=1, keepdims=True)
        pv = jax.lax.dot_general(
            p, v, (((1,), (1,)), ((), ())),
            preferred_element_type=jnp.float32)
        acc_ref[...] = acc_ref[...] * alpha + pv

        @pl.when(i == nsteps - 1)
        def _done():
            o_ref[...] = acc_ref[...] / d_ref[...]

    return body


@functools.partial(jax.jit, static_argnames=("interpret",))
def kernel(latent, keys, vals, hardness, interpret=False):
    b, l, d = latent.shape
    s = keys.shape[0]
    nq = b * l
    q = latent.reshape(nq, d)
    padded = -(-s // 128) * 128
    chunk = min(_CHUNK, padded)
    nfull = (padded - 1) // chunk
    last = padded - nfull * chunk
    nsteps = nfull + 1
    valid_last = s - nfull * chunk
    grid = (nsteps,)
    kt = keys.T
    vt = vals.T
    hp = jnp.zeros((nsteps * chunk,), jnp.float32).at[:s].set(hardness)
    h3 = hp.reshape(nsteps, 1, chunk)
    out = pl.pallas_call(
        _make_body(chunk, last, nsteps, valid_last),
        grid=grid,
        in_specs=[
            pl.BlockSpec((nq, d), lambda i: (0, 0)),
            pl.BlockSpec(memory_space=pl.ANY),
            pl.BlockSpec(memory_space=pl.ANY),
            pl.BlockSpec((1, 1, chunk), lambda i: (i, 0, 0)),
        ],
        out_specs=pl.BlockSpec((nq, d), lambda i: (0, 0)),
        out_shape=jax.ShapeDtypeStruct((nq, d), jnp.float32),
        scratch_shapes=[
            pltpu.VMEM((_NBUF, d, chunk), jnp.float32),
            pltpu.VMEM((_NBUF, d, chunk), jnp.float32),
            pltpu.VMEM((nq, d), jnp.float32),
            pltpu.VMEM((nq, 1), jnp.float32),
            pltpu.VMEM((nq, 1), jnp.float32),
            pltpu.VMEM((nq, d), jnp.float32),
            pltpu.SemaphoreType.DMA((_NBUF,)),
            pltpu.SemaphoreType.DMA((_NBUF,)),
        ],
        interpret=interpret,
    )(q, kt, vt, h3)
    return out.reshape(b, l, d)


# speedup vs baseline: 1.0667x; 1.0667x over previous
import functools

import jax
import jax.numpy as jnp
from jax.experimental import pallas as pl
from jax.experimental.pallas import tpu as pltpu

_NBUF = 4
_LOOKAHEAD = 3
_CHUNK = 8192


def _make_body(chunk, last, nsteps, valid_last):
    def body(q_ref, kt_hbm, vt_hbm, h_ref, o_ref,
             k_buf, v_buf, qn_ref, m_ref, d_ref, acc_ref, k_sem, v_sem):
        i = pl.program_id(0)

        def start(step):
            step = jnp.asarray(step, jnp.int32)
            b = jax.lax.rem(step, _NBUF)

            @pl.when(step < nsteps - 1)
            def _full():
                cols = pl.ds(step * chunk, chunk)
                pltpu.make_async_copy(
                    kt_hbm.at[:, cols], k_buf.at[b], k_sem.at[b]).start()
                pltpu.make_async_copy(
                    vt_hbm.at[:, cols], v_buf.at[b], v_sem.at[b]).start()

            @pl.when(step == nsteps - 1)
            def _tail():
                cols = pl.ds(step * chunk, last)
                dst = pl.ds(0, last)
                pltpu.make_async_copy(
                    kt_hbm.at[:, cols], k_buf.at[b, :, dst], k_sem.at[b]).start()
                pltpu.make_async_copy(
                    vt_hbm.at[:, cols], v_buf.at[b, :, dst], v_sem.at[b]).start()

        @pl.when(i == 0)
        def _init():
            for sstep in range(min(_LOOKAHEAD, nsteps)):
                start(sstep)
            q = q_ref[...]
            qn = q / jnp.maximum(
                jnp.sqrt(jnp.sum(q * q, axis=1, keepdims=True)), 1e-12)
            qn_ref[...] = qn
            m_ref[...] = jnp.full_like(m_ref, -jnp.inf)
            d_ref[...] = jnp.zeros_like(d_ref)
            acc_ref[...] = jnp.zeros_like(acc_ref)

        @pl.when(jnp.logical_and(i > 0, i + _LOOKAHEAD - 1 < nsteps))
        def _prefetch():
            start(i + _LOOKAHEAD - 1)

        b = jax.lax.rem(i, _NBUF)

        @pl.when(i < nsteps - 1)
        def _wait_full():
            cols = pl.ds(i * chunk, chunk)
            pltpu.make_async_copy(
                kt_hbm.at[:, cols], k_buf.at[b], k_sem.at[b]).wait()
            pltpu.make_async_copy(
                vt_hbm.at[:, cols], v_buf.at[b], v_sem.at[b]).wait()

        @pl.when(i == nsteps - 1)
        def _wait_tail():
            cols = pl.ds(i * chunk, last)
            dst = pl.ds(0, last)
            pltpu.make_async_copy(
                kt_hbm.at[:, cols], k_buf.at[b, :, dst], k_sem.at[b]).wait()
            pltpu.make_async_copy(
                vt_hbm.at[:, cols], v_buf.at[b, :, dst], v_sem.at[b]).wait()

        kt = k_buf[b]
        qn = qn_ref[...]
        d = qn.shape[1]
        raw = jax.lax.dot_general(
            qn, kt, (((1,), (0,)), ((), ())),
            preferred_element_type=jnp.float32)
        ones = jnp.ones((1, d), jnp.float32)
        sumsq = jax.lax.dot_general(
            ones, kt * kt, (((1,), (0,)), ((), ())),
            preferred_element_type=jnp.float32)
        inv_norm = 1.0 / jnp.maximum(jnp.sqrt(sumsq), 1e-12)
        scores = raw * (h_ref[0] * inv_norm * 10.0)
        v = v_buf[b]
        col = jax.lax.broadcasted_iota(jnp.int32, (1, chunk), 1)
        bad = jnp.logical_and(i == nsteps - 1, col >= valid_last)
        scores = jnp.where(bad, -1e30, scores)
        v = jnp.where(bad, 0.0, v)

        m_prev = m_ref[...]
        m_new = jnp.maximum(m_prev, jnp.max(scores, axis=1, keepdims=True))
        alpha = jnp.exp(m_prev - m_new)
        p = jnp.exp(scores - m_new)
        m_ref[...] = m_new
        d_ref[...] = d_ref[...] * alpha + jnp.sum(p, axis=1, keepdims=True)
        pv = jax.lax.dot_general(
            p, v, (((1,), (1,)), ((), ())),
            preferred_element_type=jnp.float32)
        acc_ref[...] = acc_ref[...] * alpha + pv

        @pl.when(i == nsteps - 1)
        def _done():
            o_ref[...] = acc_ref[...] / d_ref[...]

    return body


@functools.partial(jax.jit, static_argnames=("interpret",))
def kernel(latent, keys, vals, hardness, interpret=False):
    b, l, d = latent.shape
    s = keys.shape[0]
    nq = b * l
    q = latent.reshape(nq, d)
    padded = -(-s // 128) * 128
    chunk = min(_CHUNK, padded)
    nfull = (padded - 1) // chunk
    last = padded - nfull * chunk
    nsteps = nfull + 1
    valid_last = s - nfull * chunk
    grid = (nsteps,)
    kt = keys.T
    vt = vals.T
    hp = jnp.zeros((nsteps * chunk,), jnp.float32).at[:s].set(hardness)
    h3 = hp.reshape(nsteps, 1, chunk)
    out = pl.pallas_call(
        _make_body(chunk, last, nsteps, valid_last),
        grid=grid,
        in_specs=[
            pl.BlockSpec((nq, d), lambda i: (0, 0)),
            pl.BlockSpec(memory_space=pl.ANY),
            pl.BlockSpec(memory_space=pl.ANY),
            pl.BlockSpec((1, 1, chunk), lambda i: (i, 0, 0)),
        ],
        out_specs=pl.BlockSpec((nq, d), lambda i: (0, 0)),
        out_shape=jax.ShapeDtypeStruct((nq, d), jnp.float32),
        scratch_shapes=[
            pltpu.VMEM((_NBUF, d, chunk), jnp.float32),
            pltpu.VMEM((_NBUF, d, chunk), jnp.float32),
            pltpu.VMEM((nq, d), jnp.float32),
            pltpu.VMEM((nq, 1), jnp.float32),
            pltpu.VMEM((nq, 1), jnp.float32),
            pltpu.VMEM((nq, d), jnp.float32),
            pltpu.SemaphoreType.DMA((_NBUF,)),
            pltpu.SemaphoreType.DMA((_NBUF,)),
        ],
        interpret=interpret,
    )(q, kt, vt, h3)
    return out.reshape(b, l, d)


# speedup vs baseline: 1.1997x; 1.1247x over previous
import functools

import jax
import jax.numpy as jnp
from jax.experimental import pallas as pl
from jax.experimental.pallas import tpu as pltpu

_NBUF = 4
_LOOKAHEAD = 3
_CHUNK = 10240


def _make_body(chunk, last, nsteps, valid_last, h_last):
    def body(q_ref, kt_hbm, vt_hbm, h_hbm, o_ref,
             k_buf, v_buf, h_buf, qn_ref, m_ref, d_ref, acc_ref,
             k_sem, v_sem, h_sem):
        i = pl.program_id(0)

        def start(step):
            step = jnp.asarray(step, jnp.int32)
            b = jax.lax.rem(step, _NBUF)

            @pl.when(step < nsteps - 1)
            def _full():
                cols = pl.ds(step * chunk, chunk)
                pltpu.make_async_copy(
                    kt_hbm.at[:, cols], k_buf.at[b], k_sem.at[b]).start()
                pltpu.make_async_copy(
                    vt_hbm.at[:, cols], v_buf.at[b], v_sem.at[b]).start()
                pltpu.make_async_copy(
                    h_hbm.at[pl.ds(step * chunk, chunk)],
                    h_buf.at[b, 0], h_sem.at[b]).start()

            @pl.when(step == nsteps - 1)
            def _tail():
                cols = pl.ds(step * chunk, last)
                dst = pl.ds(0, last)
                pltpu.make_async_copy(
                    kt_hbm.at[:, cols], k_buf.at[b, :, dst], k_sem.at[b]).start()
                pltpu.make_async_copy(
                    vt_hbm.at[:, cols], v_buf.at[b, :, dst], v_sem.at[b]).start()
                pltpu.make_async_copy(
                    h_hbm.at[pl.ds(step * chunk, h_last)],
                    h_buf.at[b, 0, pl.ds(0, h_last)], h_sem.at[b]).start()

        @pl.when(i == 0)
        def _init():
            for sstep in range(min(_LOOKAHEAD, nsteps)):
                start(sstep)
            q = q_ref[...]
            qn = q / jnp.maximum(
                jnp.sqrt(jnp.sum(q * q, axis=1, keepdims=True)), 1e-12)
            qn_ref[...] = qn
            m_ref[...] = jnp.full_like(m_ref, -jnp.inf)
            d_ref[...] = jnp.zeros_like(d_ref)
            acc_ref[...] = jnp.zeros_like(acc_ref)

        @pl.when(jnp.logical_and(i > 0, i + _LOOKAHEAD - 1 < nsteps))
        def _prefetch():
            start(i + _LOOKAHEAD - 1)

        b = jax.lax.rem(i, _NBUF)

        @pl.when(i < nsteps - 1)
        def _wait_full():
            cols = pl.ds(i * chunk, chunk)
            pltpu.make_async_copy(
                kt_hbm.at[:, cols], k_buf.at[b], k_sem.at[b]).wait()
            pltpu.make_async_copy(
                vt_hbm.at[:, cols], v_buf.at[b], v_sem.at[b]).wait()
            pltpu.make_async_copy(
                h_hbm.at[pl.ds(i * chunk, chunk)],
                h_buf.at[b, 0], h_sem.at[b]).wait()

        @pl.when(i == nsteps - 1)
        def _wait_tail():
            cols = pl.ds(i * chunk, last)
            dst = pl.ds(0, last)
            pltpu.make_async_copy(
                kt_hbm.at[:, cols], k_buf.at[b, :, dst], k_sem.at[b]).wait()
            pltpu.make_async_copy(
                vt_hbm.at[:, cols], v_buf.at[b, :, dst], v_sem.at[b]).wait()
            pltpu.make_async_copy(
                h_hbm.at[pl.ds(i * chunk, h_last)],
                h_buf.at[b, 0, pl.ds(0, h_last)], h_sem.at[b]).wait()

        kt = k_buf[b]
        qn = qn_ref[...]
        d = qn.shape[1]
        raw = jax.lax.dot_general(
            qn, kt, (((1,), (0,)), ((), ())),
            preferred_element_type=jnp.float32)
        ones = jnp.ones((1, d), jnp.float32)
        sumsq = jax.lax.dot_general(
            ones, kt * kt, (((1,), (0,)), ((), ())),
            preferred_element_type=jnp.float32)
        inv_norm = 1.0 / jnp.maximum(jnp.sqrt(sumsq), 1e-12)
        scores = raw * (h_buf[b] * inv_norm * 10.0)
        v = v_buf[b]
        col = jax.lax.broadcasted_iota(jnp.int32, (1, chunk), 1)
        bad = jnp.logical_and(i == nsteps - 1, col >= valid_last)
        scores = jnp.where(bad, -1e30, scores)
        v = jnp.where(bad, 0.0, v)

        m_prev = m_ref[...]
        m_new = jnp.maximum(m_prev, jnp.max(scores, axis=1, keepdims=True))
        alpha = jnp.exp(m_prev - m_new)
        p = jnp.exp(scores - m_new)
        m_ref[...] = m_new
        d_ref[...] = d_ref[...] * alpha + jnp.sum(p, axis=1, keepdims=True)
        pv = jax.lax.dot_general(
            p, v, (((1,), (1,)), ((), ())),
            preferred_element_type=jnp.float32)
        acc_ref[...] = acc_ref[...] * alpha + pv

        @pl.when(i == nsteps - 1)
        def _done():
            o_ref[...] = acc_ref[...] / d_ref[...]

    return body


@functools.partial(jax.jit, static_argnames=("interpret",))
def kernel(latent, keys, vals, hardness, interpret=False):
    b, l, d = latent.shape
    s = keys.shape[0]
    nq = b * l
    q = latent.reshape(nq, d)
    padded = -(-s // 128) * 128
    chunk = min(_CHUNK, padded)
    nfull = (padded - 1) // chunk
    last = padded - nfull * chunk
    nsteps = nfull + 1
    valid_last = s - nfull * chunk
    h_pad = -(-s // 1024) * 1024
    h_last = min(h_pad - nfull * chunk, chunk)
    grid = (nsteps,)
    kt = keys.T
    vt = vals.T
    out = pl.pallas_call(
        _make_body(chunk, last, nsteps, valid_last, h_last),
        grid=grid,
        in_specs=[
            pl.BlockSpec((nq, d), lambda i: (0, 0)),
            pl.BlockSpec(memory_space=pl.ANY),
            pl.BlockSpec(memory_space=pl.ANY),
            pl.BlockSpec(memory_space=pl.ANY),
        ],
        out_specs=pl.BlockSpec((nq, d), lambda i: (0, 0)),
        out_shape=jax.ShapeDtypeStruct((nq, d), jnp.float32),
        scratch_shapes=[
            pltpu.VMEM((_NBUF, d, chunk), jnp.float32),
            pltpu.VMEM((_NBUF, d, chunk), jnp.float32),
            pltpu.VMEM((_NBUF, 1, chunk), jnp.float32),
            pltpu.VMEM((nq, d), jnp.float32),
            pltpu.VMEM((nq, 1), jnp.float32),
            pltpu.VMEM((nq, 1), jnp.float32),
            pltpu.VMEM((nq, d), jnp.float32),
            pltpu.SemaphoreType.DMA((_NBUF,)),
            pltpu.SemaphoreType.DMA((_NBUF,)),
            pltpu.SemaphoreType.DMA((_NBUF,)),
        ],
        interpret=interpret,
    )(q, kt, vt, hardness)
    return out.reshape(b, l, d)
